# baseline (speedup 1.0000x reference)
import jax
import jax.numpy as jnp
from jax import lax
from jax.experimental import pallas as pl
from jax.experimental.pallas import tpu as pltpu
from jax.experimental.pallas import tpu_sc as plsc

L = 16
NCHUNK = 7
NPAD = NCHUNK * L
NBOX = 100
NIMG = 8
IOU_THR = 0.5
SORT_W = 5 * NPAD
ACW = NPAD + L


def _nms_body(sc_hbm, bx_hbm, bxo_hbm, spk_hbm,
              s_v, b_v, o_v, area_v, keep_v,
              acx1_v, acy1_v, acx2_v, acy2_v, acar_v, acidx_v, keepc_v,
              os_v, obox_v):
    wid = lax.axis_index("s") * 2 + lax.axis_index("c")

    @pl.when(wid < NIMG)
    def _():
        pltpu.sync_copy(sc_hbm.at[wid], s_v)
        pltpu.sync_copy(bx_hbm.at[wid], b_v)

        iota = lax.iota(jnp.int32, L)
        gidx = [iota + iv * L for iv in range(NCHUNK)]
        zeros = jnp.zeros((L,), jnp.int32)
        svecs = [s_v[pl.ds(iv * L, L)] for iv in range(NCHUNK)]

        def rank_body(jq, ranks):
            j0 = jq * 4
            sjs = [plsc.load_gather(s_v, [zeros + (j0 + u)]) for u in range(4)]
            out = []
            for iv in range(NCHUNK):
                acc = ranks[iv]
                for u in range(4):
                    beats = ((sjs[u] > svecs[iv]) |
                             ((sjs[u] == svecs[iv]) & (j0 + u < gidx[iv])))
                    acc = acc + beats.astype(jnp.int32)
                out.append(acc)
            return tuple(out)

        ranks = lax.fori_loop(0, NBOX // 4, rank_body,
                              tuple(zeros for _ in range(NCHUNK)))
        last = NCHUNK - 1
        ranks = ranks[:last] + (
            jnp.where(gidx[last] >= NBOX, gidx[last], ranks[last]),)

        for iv in range(NCHUNK):
            r = ranks[iv]
            plsc.store_scatter(o_v, [r], svecs[iv])
            rows4 = jnp.minimum(gidx[iv], NBOX - 1) * 4
            inb = gidx[iv] < NBOX
            for k in range(4):
                c = plsc.load_gather(b_v, [rows4 + k])
                c = jnp.where(inb, c, 0.0)
                plsc.store_scatter(o_v, [r + (k + 1) * NPAD], c)

        ones = jnp.ones((L,), jnp.int32)
        n_act = jnp.int32(0)
        for iv in range(NCHUNK):
            sl = pl.ds(iv * L, L)
            x1c = o_v[pl.ds(1 * NPAD + iv * L, L)]
            y1c = o_v[pl.ds(2 * NPAD + iv * L, L)]
            x2c = o_v[pl.ds(3 * NPAD + iv * L, L)]
            y2c = o_v[pl.ds(4 * NPAD + iv * L, L)]
            ar = (jnp.maximum(x2c - x1c, 0.0) *
                  jnp.maximum(y2c - y1c, 0.0))
            area_v[sl] = ar
            keep_v[sl] = ones
            act = (x2c > x1c) & (y2c > y1c)
            dst = pl.ds(n_act, L)
            plsc.store_compressed(acx1_v.at[dst], x1c, mask=act)
            plsc.store_compressed(acy1_v.at[dst], y1c, mask=act)
            plsc.store_compressed(acx2_v.at[dst], x2c, mask=act)
            plsc.store_compressed(acy2_v.at[dst], y2c, mask=act)
            plsc.store_compressed(acar_v.at[dst], ar, mask=act)
            plsc.store_compressed(acidx_v.at[dst], gidx[iv], mask=act)
            n_act = n_act + jnp.sum(act.astype(jnp.int32))
        for iv in range(NCHUNK + 1):
            keepc_v[pl.ds(iv * L, L)] = ones

        def nms_body(t, carry):
            ts = zeros + t
            alive = plsc.load_gather(keepc_v, [ts]) != 0
            xi1 = plsc.load_gather(acx1_v, [ts])
            yi1 = plsc.load_gather(acy1_v, [ts])
            xi2 = plsc.load_gather(acx2_v, [ts])
            yi2 = plsc.load_gather(acy2_v, [ts])
            ai = plsc.load_gather(acar_v, [ts])
            for jc in range(NCHUNK):
                @pl.when((jc * L < n_act) & (jc * L + (L - 1) > t))
                def _(jc=jc):
                    sl = pl.ds(jc * L, L)
                    xx1 = jnp.maximum(acx1_v[sl], xi1)
                    yy1 = jnp.maximum(acy1_v[sl], yi1)
                    xx2 = jnp.minimum(acx2_v[sl], xi2)
                    yy2 = jnp.minimum(acy2_v[sl], yi2)
                    inter = (jnp.maximum(xx2 - xx1, 0.0) *
                             jnp.maximum(yy2 - yy1, 0.0))
                    union = ai + acar_v[sl] - inter
                    iou = inter / jnp.maximum(union, 1e-9)
                    supp = (iou > IOU_THR) & (gidx[jc] > t) & alive
                    keepc_v[sl] = jnp.where(supp, 0, keepc_v[sl])
            return carry

        lax.fori_loop(0, n_act, nms_body, 0)

        for jc in range(NCHUNK):
            @pl.when(jc * L < n_act)
            def _(jc=jc):
                sl = pl.ds(jc * L, L)
                plsc.store_scatter(keep_v, [acidx_v[sl]], keepc_v[sl],
                                   mask=gidx[jc] < n_act)

        total = jnp.int32(0)
        for iv in range(NCHUNK):
            sl = pl.ds(iv * L, L)
            kv = keep_v[sl]
            total = total + jnp.sum(kv * (gidx[iv] < NBOX).astype(jnp.int32))
            kf = kv.astype(jnp.float32)
            os_v[sl] = o_v[sl] * kf
            rows4 = jnp.minimum(gidx[iv], NBOX - 1) * 4
            inb = gidx[iv] < NBOX
            for k in range(4):
                bm = o_v[pl.ds((k + 1) * NPAD + iv * L, L)] * kf
                plsc.store_scatter(obox_v, [rows4 + k], bm, mask=inb)
        tail = os_v[pl.ds(NPAD - L, L)]
        tail = jnp.where(gidx[NCHUNK - 1] < NBOX, tail,
                         total.astype(jnp.float32))
        os_v[pl.ds(NPAD - L, L)] = tail

        pltpu.sync_copy(obox_v, bxo_hbm.at[wid])
        pltpu.sync_copy(os_v, spk_hbm.at[wid])


def kernel(x, boxes, scores, W1, b1, W2, b2, Wb, bb, Wc, bc):
    del x, W1, b1, W2, b2, Wb, bb, Wc, bc
    nb, nn = scores.shape
    sc_p = jnp.pad(scores, ((0, 0), (0, NPAD - nn)), constant_values=-1.0)
    bx_flat = boxes.reshape(nb, nn * 4)

    mesh = plsc.VectorSubcoreMesh(core_axis_name="c", subcore_axis_name="s",
                                  num_cores=2, num_subcores=16)
    f32 = jnp.float32
    bxo, spk = pl.kernel(
        _nms_body,
        out_type=(
            jax.ShapeDtypeStruct((nb, nn * 4), f32),
            jax.ShapeDtypeStruct((nb, NPAD), f32),
        ),
        mesh=mesh,
        compiler_params=pltpu.CompilerParams(needs_layout_passes=False),
        scratch_types=[
            pltpu.VMEM((NPAD,), f32),
            pltpu.VMEM((NBOX * 4,), f32),
            pltpu.VMEM((SORT_W,), f32),
            pltpu.VMEM((NPAD,), f32),
            pltpu.VMEM((NPAD,), jnp.int32),
            pltpu.VMEM((ACW,), f32),
            pltpu.VMEM((ACW,), f32),
            pltpu.VMEM((ACW,), f32),
            pltpu.VMEM((ACW,), f32),
            pltpu.VMEM((ACW,), f32),
            pltpu.VMEM((ACW,), jnp.int32),
            pltpu.VMEM((ACW,), jnp.int32),
            pltpu.VMEM((NPAD,), f32),
            pltpu.VMEM((NBOX * 4,), f32),
        ],
    )(sc_p, bx_flat)

    final_boxes = bxo.reshape(nb, nn, 4)
    final_scores = spk[:, :nn]
    num_detections = spk[:, nn].astype(jnp.int32)
    return final_boxes, final_scores, num_detections


# speedup vs baseline: 1.2857x; 1.2857x over previous
import jax
import jax.numpy as jnp
from jax import lax
from jax.experimental import pallas as pl
from jax.experimental.pallas import tpu as pltpu
from jax.experimental.pallas import tpu_sc as plsc

L = 16
NCHUNK = 7
NPAD = NCHUNK * L
NBOX = 100
NIMG = 8
IOU_THR = 0.5
SORT_W = 5 * NPAD
ACW = NPAD + L


def _nms_body(sc_hbm, bx_hbm, bxo_hbm, spk_hbm,
              s_v, b_v, o_v, area_v, keep_v,
              acx1_v, acy1_v, acx2_v, acy2_v, acar_v, acidx_v, keepc_v,
              os_v, obox_v):
    wid = lax.axis_index("s") * 2 + lax.axis_index("c")

    @pl.when(wid < NIMG)
    def _():
        pltpu.sync_copy(sc_hbm.at[wid], s_v)
        pltpu.sync_copy(bx_hbm.at[wid], b_v)

        iota = lax.iota(jnp.int32, L)
        gidx = [iota + iv * L for iv in range(NCHUNK)]
        zeros = jnp.zeros((L,), jnp.int32)
        svecs = [s_v[pl.ds(iv * L, L)] for iv in range(NCHUNK)]

        def rank_body(j, ranks):
            sj = plsc.load_gather(s_v, [zeros + j])
            out = []
            for iv in range(NCHUNK):
                beats = (sj > svecs[iv]) | ((sj == svecs[iv]) & (j < gidx[iv]))
                out.append(ranks[iv] + beats.astype(jnp.int32))
            return tuple(out)

        ranks = lax.fori_loop(0, NBOX, rank_body,
                              tuple(zeros for _ in range(NCHUNK)))
        last = NCHUNK - 1
        ranks = ranks[:last] + (
            jnp.where(gidx[last] >= NBOX, gidx[last], ranks[last]),)

        for iv in range(NCHUNK):
            r = ranks[iv]
            plsc.store_scatter(o_v, [r], svecs[iv])
            rows4 = jnp.minimum(gidx[iv], NBOX - 1) * 4
            inb = gidx[iv] < NBOX
            for k in range(4):
                c = plsc.load_gather(b_v, [rows4 + k])
                c = jnp.where(inb, c, 0.0)
                plsc.store_scatter(o_v, [r + (k + 1) * NPAD], c)

        ones = jnp.ones((L,), jnp.int32)
        n_act = jnp.int32(0)
        for iv in range(NCHUNK):
            sl = pl.ds(iv * L, L)
            x1c = o_v[pl.ds(1 * NPAD + iv * L, L)]
            y1c = o_v[pl.ds(2 * NPAD + iv * L, L)]
            x2c = o_v[pl.ds(3 * NPAD + iv * L, L)]
            y2c = o_v[pl.ds(4 * NPAD + iv * L, L)]
            ar = (jnp.maximum(x2c - x1c, 0.0) *
                  jnp.maximum(y2c - y1c, 0.0))
            area_v[sl] = ar
            keep_v[sl] = ones
            act = (x2c > x1c) & (y2c > y1c)
            dst = pl.ds(n_act, L)
            plsc.store_compressed(acx1_v.at[dst], x1c, mask=act)
            plsc.store_compressed(acy1_v.at[dst], y1c, mask=act)
            plsc.store_compressed(acx2_v.at[dst], x2c, mask=act)
            plsc.store_compressed(acy2_v.at[dst], y2c, mask=act)
            plsc.store_compressed(acar_v.at[dst], ar, mask=act)
            plsc.store_compressed(acidx_v.at[dst], gidx[iv], mask=act)
            n_act = n_act + jnp.sum(act.astype(jnp.int32))
        for iv in range(NCHUNK + 1):
            keepc_v[pl.ds(iv * L, L)] = ones

        def nms_body(t, carry):
            ts = zeros + t
            alive = plsc.load_gather(keepc_v, [ts]) != 0
            xi1 = plsc.load_gather(acx1_v, [ts])
            yi1 = plsc.load_gather(acy1_v, [ts])
            xi2 = plsc.load_gather(acx2_v, [ts])
            yi2 = plsc.load_gather(acy2_v, [ts])
            ai = plsc.load_gather(acar_v, [ts])
            for jc in range(NCHUNK):
                @pl.when((jc * L < n_act) & (jc * L + (L - 1) > t))
                def _(jc=jc):
                    sl = pl.ds(jc * L, L)
                    xx1 = jnp.maximum(acx1_v[sl], xi1)
                    yy1 = jnp.maximum(acy1_v[sl], yi1)
                    xx2 = jnp.minimum(acx2_v[sl], xi2)
                    yy2 = jnp.minimum(acy2_v[sl], yi2)
                    inter = (jnp.maximum(xx2 - xx1, 0.0) *
                             jnp.maximum(yy2 - yy1, 0.0))
                    union = ai + acar_v[sl] - inter
                    iou = inter / jnp.maximum(union, 1e-9)
                    supp = (iou > IOU_THR) & (gidx[jc] > t) & alive
                    keepc_v[sl] = jnp.where(supp, 0, keepc_v[sl])
            return carry

        lax.fori_loop(0, n_act, nms_body, 0)

        for jc in range(NCHUNK):
            @pl.when(jc * L < n_act)
            def _(jc=jc):
                sl = pl.ds(jc * L, L)
                plsc.store_scatter(keep_v, [acidx_v[sl]], keepc_v[sl],
                                   mask=gidx[jc] < n_act)

        total = jnp.int32(0)
        for iv in range(NCHUNK):
            sl = pl.ds(iv * L, L)
            kv = keep_v[sl]
            total = total + jnp.sum(kv * (gidx[iv] < NBOX).astype(jnp.int32))
            kf = kv.astype(jnp.float32)
            os_v[sl] = o_v[sl] * kf
            rows4 = jnp.minimum(gidx[iv], NBOX - 1) * 4
            inb = gidx[iv] < NBOX
            for k in range(4):
                bm = o_v[pl.ds((k + 1) * NPAD + iv * L, L)] * kf
                plsc.store_scatter(obox_v, [rows4 + k], bm, mask=inb)
        tail = os_v[pl.ds(NPAD - L, L)]
        tail = jnp.where(gidx[NCHUNK - 1] < NBOX, tail,
                         total.astype(jnp.float32))
        os_v[pl.ds(NPAD - L, L)] = tail

        pltpu.sync_copy(obox_v, bxo_hbm.at[wid])
        pltpu.sync_copy(os_v, spk_hbm.at[wid])


def kernel(x, boxes, scores, W1, b1, W2, b2, Wb, bb, Wc, bc):
    del x, W1, b1, W2, b2, Wb, bb, Wc, bc
    nb, nn = scores.shape
    sc_p = jnp.pad(scores, ((0, 0), (0, NPAD - nn)), constant_values=-1.0)
    bx_flat = boxes.reshape(nb, nn * 4)

    mesh = plsc.VectorSubcoreMesh(core_axis_name="c", subcore_axis_name="s",
                                  num_cores=2, num_subcores=16)
    f32 = jnp.float32
    bxo, spk = pl.kernel(
        _nms_body,
        out_type=(
            jax.ShapeDtypeStruct((nb, nn * 4), f32),
            jax.ShapeDtypeStruct((nb, NPAD), f32),
        ),
        mesh=mesh,
        compiler_params=pltpu.CompilerParams(needs_layout_passes=False),
        scratch_types=[
            pltpu.VMEM((NPAD,), f32),
            pltpu.VMEM((NBOX * 4,), f32),
            pltpu.VMEM((SORT_W,), f32),
            pltpu.VMEM((NPAD,), f32),
            pltpu.VMEM((NPAD,), jnp.int32),
            pltpu.VMEM((ACW,), f32),
            pltpu.VMEM((ACW,), f32),
            pltpu.VMEM((ACW,), f32),
            pltpu.VMEM((ACW,), f32),
            pltpu.VMEM((ACW,), f32),
            pltpu.VMEM((ACW,), jnp.int32),
            pltpu.VMEM((ACW,), jnp.int32),
            pltpu.VMEM((NPAD,), f32),
            pltpu.VMEM((NBOX * 4,), f32),
        ],
    )(sc_p, bx_flat)

    final_boxes = bxo.reshape(nb, nn, 4)
    final_scores = spk[:, :nn]
    num_detections = spk[:, nn].astype(jnp.int32)
    return final_boxes, final_scores, num_detections


# speedup vs baseline: 1.3630x; 1.0601x over previous
import jax
import jax.numpy as jnp
from jax import lax
from jax.experimental import pallas as pl
from jax.experimental.pallas import tpu as pltpu
from jax.experimental.pallas import tpu_sc as plsc

L = 16
NCHUNK = 7
NPAD = NCHUNK * L
NBOX = 100
NIMG = 8
IOU_THR = 0.5
SORT_W = 5 * NPAD
ACW = NPAD + L


def _nms_body(sc_hbm, bx_hbm, bxo_hbm, spk_hbm,
              s_v, b_v, o_v, area_v, keep_v,
              acx1_v, acy1_v, acx2_v, acy2_v, acar_v, acidx_v, keepc_v,
              os_v, obox_v):
    wid = lax.axis_index("s")

    @pl.when(wid < NIMG)
    def _():
        pltpu.sync_copy(sc_hbm.at[wid], s_v)
        pltpu.sync_copy(bx_hbm.at[wid], b_v)

        iota = lax.iota(jnp.int32, L)
        gidx = [iota + iv * L for iv in range(NCHUNK)]
        zeros = jnp.zeros((L,), jnp.int32)
        svecs = [s_v[pl.ds(iv * L, L)] for iv in range(NCHUNK)]

        def rank_body(j, ranks):
            sj = plsc.load_gather(s_v, [zeros + j])
            out = []
            for iv in range(NCHUNK):
                beats = (sj > svecs[iv]) | ((sj == svecs[iv]) & (j < gidx[iv]))
                out.append(ranks[iv] + beats.astype(jnp.int32))
            return tuple(out)

        ranks = lax.fori_loop(0, NBOX, rank_body,
                              tuple(zeros for _ in range(NCHUNK)))
        last = NCHUNK - 1
        ranks = ranks[:last] + (
            jnp.where(gidx[last] >= NBOX, gidx[last], ranks[last]),)

        for iv in range(NCHUNK):
            r = ranks[iv]
            plsc.store_scatter(o_v, [r], svecs[iv])
            rows4 = jnp.minimum(gidx[iv], NBOX - 1) * 4
            inb = gidx[iv] < NBOX
            for k in range(4):
                c = plsc.load_gather(b_v, [rows4 + k])
                c = jnp.where(inb, c, 0.0)
                plsc.store_scatter(o_v, [r + (k + 1) * NPAD], c)

        ones = jnp.ones((L,), jnp.int32)
        n_act = jnp.int32(0)
        for iv in range(NCHUNK):
            sl = pl.ds(iv * L, L)
            x1c = o_v[pl.ds(1 * NPAD + iv * L, L)]
            y1c = o_v[pl.ds(2 * NPAD + iv * L, L)]
            x2c = o_v[pl.ds(3 * NPAD + iv * L, L)]
            y2c = o_v[pl.ds(4 * NPAD + iv * L, L)]
            ar = (jnp.maximum(x2c - x1c, 0.0) *
                  jnp.maximum(y2c - y1c, 0.0))
            area_v[sl] = ar
            keep_v[sl] = ones
            act = (x2c > x1c) & (y2c > y1c)
            dst = pl.ds(n_act, L)
            plsc.store_compressed(acx1_v.at[dst], x1c, mask=act)
            plsc.store_compressed(acy1_v.at[dst], y1c, mask=act)
            plsc.store_compressed(acx2_v.at[dst], x2c, mask=act)
            plsc.store_compressed(acy2_v.at[dst], y2c, mask=act)
            plsc.store_compressed(acar_v.at[dst], ar, mask=act)
            plsc.store_compressed(acidx_v.at[dst], gidx[iv], mask=act)
            n_act = n_act + jnp.sum(act.astype(jnp.int32))
        for iv in range(NCHUNK + 1):
            keepc_v[pl.ds(iv * L, L)] = ones

        def nms_body(t, carry):
            ts = zeros + t
            alive = plsc.load_gather(keepc_v, [ts]) != 0
            xi1 = plsc.load_gather(acx1_v, [ts])
            yi1 = plsc.load_gather(acy1_v, [ts])
            xi2 = plsc.load_gather(acx2_v, [ts])
            yi2 = plsc.load_gather(acy2_v, [ts])
            ai = plsc.load_gather(acar_v, [ts])
            for jc in range(NCHUNK):
                @pl.when((jc * L < n_act) & (jc * L + (L - 1) > t))
                def _(jc=jc):
                    sl = pl.ds(jc * L, L)
                    xx1 = jnp.maximum(acx1_v[sl], xi1)
                    yy1 = jnp.maximum(acy1_v[sl], yi1)
                    xx2 = jnp.minimum(acx2_v[sl], xi2)
                    yy2 = jnp.minimum(acy2_v[sl], yi2)
                    inter = (jnp.maximum(xx2 - xx1, 0.0) *
                             jnp.maximum(yy2 - yy1, 0.0))
                    union = ai + acar_v[sl] - inter
                    iou = inter / jnp.maximum(union, 1e-9)
                    supp = (iou > IOU_THR) & (gidx[jc] > t) & alive
                    keepc_v[sl] = jnp.where(supp, 0, keepc_v[sl])
            return carry

        lax.fori_loop(0, n_act, nms_body, 0)

        for jc in range(NCHUNK):
            @pl.when(jc * L < n_act)
            def _(jc=jc):
                sl = pl.ds(jc * L, L)
                plsc.store_scatter(keep_v, [acidx_v[sl]], keepc_v[sl],
                                   mask=gidx[jc] < n_act)

        total = jnp.int32(0)
        for iv in range(NCHUNK):
            sl = pl.ds(iv * L, L)
            kv = keep_v[sl]
            total = total + jnp.sum(kv * (gidx[iv] < NBOX).astype(jnp.int32))
            kf = kv.astype(jnp.float32)
            os_v[sl] = o_v[sl] * kf
            rows4 = jnp.minimum(gidx[iv], NBOX - 1) * 4
            inb = gidx[iv] < NBOX
            for k in range(4):
                bm = o_v[pl.ds((k + 1) * NPAD + iv * L, L)] * kf
                plsc.store_scatter(obox_v, [rows4 + k], bm, mask=inb)
        tail = os_v[pl.ds(NPAD - L, L)]
        tail = jnp.where(gidx[NCHUNK - 1] < NBOX, tail,
                         total.astype(jnp.float32))
        os_v[pl.ds(NPAD - L, L)] = tail

        pltpu.sync_copy(obox_v, bxo_hbm.at[wid])
        pltpu.sync_copy(os_v, spk_hbm.at[wid])


def kernel(x, boxes, scores, W1, b1, W2, b2, Wb, bb, Wc, bc):
    del x, W1, b1, W2, b2, Wb, bb, Wc, bc
    nb, nn = scores.shape
    sc_p = jnp.pad(scores, ((0, 0), (0, NPAD - nn)), constant_values=-1.0)
    bx_flat = boxes.reshape(nb, nn * 4)

    mesh = plsc.VectorSubcoreMesh(core_axis_name="c", subcore_axis_name="s",
                                  num_cores=1, num_subcores=16)
    f32 = jnp.float32
    bxo, spk = pl.kernel(
        _nms_body,
        out_type=(
            jax.ShapeDtypeStruct((nb, nn * 4), f32),
            jax.ShapeDtypeStruct((nb, NPAD), f32),
        ),
        mesh=mesh,
        compiler_params=pltpu.CompilerParams(needs_layout_passes=False),
        scratch_types=[
            pltpu.VMEM((NPAD,), f32),
            pltpu.VMEM((NBOX * 4,), f32),
            pltpu.VMEM((SORT_W,), f32),
            pltpu.VMEM((NPAD,), f32),
            pltpu.VMEM((NPAD,), jnp.int32),
            pltpu.VMEM((ACW,), f32),
            pltpu.VMEM((ACW,), f32),
            pltpu.VMEM((ACW,), f32),
            pltpu.VMEM((ACW,), f32),
            pltpu.VMEM((ACW,), f32),
            pltpu.VMEM((ACW,), jnp.int32),
            pltpu.VMEM((ACW,), jnp.int32),
            pltpu.VMEM((NPAD,), f32),
            pltpu.VMEM((NBOX * 4,), f32),
        ],
    )(sc_p, bx_flat)

    final_boxes = bxo.reshape(nb, nn, 4)
    final_scores = spk[:, :nn]
    num_detections = spk[:, nn].astype(jnp.int32)
    return final_boxes, final_scores, num_detections
